# baseline (speedup 1.0000x reference)
import functools

import jax
import jax.numpy as jnp
from jax import lax
from jax.experimental import pallas as pl
from jax.experimental.pallas import tpu as pltpu
from jax.experimental.pallas import tpu_sc as plsc

_NW = 8


def _sc_gather(B, D, V):
    info = plsc.get_sparse_core_info()
    nc = info.num_cores
    bpw = B // _NW
    mesh = plsc.VectorSubcoreMesh(core_axis_name="c", subcore_axis_name="s")

    @functools.partial(
        pl.kernel,
        mesh=mesh,
        out_type=jax.ShapeDtypeStruct((B, D), jnp.float32),
        scratch_types=[
            pltpu.VMEM((bpw,), jnp.int32),
            pltpu.VMEM((bpw, D), jnp.float32),
            pltpu.SemaphoreType.DMA,
        ],
    )
    def gather(w_hbm, idx_hbm, out_hbm, idx_v, rows_v, sem):
        wid = lax.axis_index("s") * nc + lax.axis_index("c")

        @pl.when(wid < _NW)
        def _():
            base = wid * bpw
            pltpu.sync_copy(idx_hbm.at[pl.ds(base, bpw)], idx_v)
            pltpu.async_copy(w_hbm.at[idx_v], rows_v, sem).wait()
            pltpu.sync_copy(rows_v, out_hbm.at[pl.ds(base, bpw)])

    return gather


def _concat_body(emb_ref, feat_ref, out_ref):
    b = pl.program_id(0)
    out_ref[0, 0, :] = emb_ref[b, :]
    out_ref[0, 1:, :] = feat_ref[0]


def kernel(feature, index_value, embedding_offset, W):
    B, S, D = feature.shape
    V = W.shape[0]
    if isinstance(embedding_offset, int) and embedding_offset == 0:
        idx = index_value.astype(jnp.int32)
    else:
        idx = (index_value - embedding_offset).astype(jnp.int32)
    emb = _sc_gather(B, D, V)(W, idx)
    grid_spec = pl.GridSpec(
        grid=(B,),
        in_specs=[
            pl.BlockSpec((B, D), lambda b: (0, 0)),
            pl.BlockSpec((1, S, D), lambda b: (b, 0, 0)),
        ],
        out_specs=pl.BlockSpec((1, S + 1, D), lambda b: (b, 0, 0)),
    )
    return pl.pallas_call(
        _concat_body,
        grid_spec=grid_spec,
        out_shape=jax.ShapeDtypeStruct((B, S + 1, D), feature.dtype),
    )(emb, feature)


# speedup vs baseline: 1.0059x; 1.0059x over previous
import functools

import jax
import jax.numpy as jnp
from jax import lax
from jax.experimental import pallas as pl
from jax.experimental.pallas import tpu as pltpu
from jax.experimental.pallas import tpu_sc as plsc

_NW = 8
_BB = 4
_SC_CHUNKS = 2


def _sc_gather(B, D):
    info = plsc.get_sparse_core_info()
    nc = info.num_cores
    bpw = B // _NW
    mesh = plsc.VectorSubcoreMesh(core_axis_name="c", subcore_axis_name="s")

    @functools.partial(
        pl.kernel,
        mesh=mesh,
        out_type=jax.ShapeDtypeStruct((B, D), jnp.float32),
        scratch_types=[
            pltpu.VMEM((bpw,), jnp.int32),
            pltpu.VMEM((bpw, D), jnp.float32),
            pltpu.SemaphoreType.DMA,
        ],
    )
    def gather(w_hbm, idx_hbm, out_hbm, idx_v, rows_v, sem):
        wid = lax.axis_index("s") * nc + lax.axis_index("c")

        @pl.when(wid < _NW)
        def _():
            base = wid * bpw
            pltpu.sync_copy(idx_hbm.at[pl.ds(base, bpw)], idx_v)
            pltpu.async_copy(w_hbm.at[idx_v], rows_v, sem).wait()
            pltpu.sync_copy(rows_v, out_hbm.at[pl.ds(base, bpw)])

    return gather


def _concat_body(emb_ref, feat_ref, out_ref):
    g = pl.program_id(0)
    c = pl.program_id(1)
    chunk = feat_ref.shape[1]

    @pl.when(c == 0)
    def _emb_rows():
        for j in range(_BB):
            out_ref[j, 0, :] = emb_ref[g * _BB + j, :]

    for cc in range(_SC_CHUNKS):

        @pl.when(c == cc)
        def _store_chunk(cc=cc):
            for j in range(_BB):
                out_ref[j, pl.ds(1 + cc * chunk, chunk), :] = feat_ref[j]


def kernel(feature, index_value, embedding_offset, W):
    B, S, D = feature.shape
    if isinstance(embedding_offset, int) and embedding_offset == 0:
        idx = index_value.astype(jnp.int32)
    else:
        idx = (index_value - embedding_offset).astype(jnp.int32)
    emb = _sc_gather(B, D)(W, idx)
    grid_spec = pl.GridSpec(
        grid=(B // _BB, _SC_CHUNKS),
        in_specs=[
            pl.BlockSpec((B, D), lambda g, c: (0, 0)),
            pl.BlockSpec((_BB, S // _SC_CHUNKS, D), lambda g, c: (g, c, 0)),
        ],
        out_specs=pl.BlockSpec((_BB, S + 1, D), lambda g, c: (g, 0, 0)),
    )
    return pl.pallas_call(
        _concat_body,
        grid_spec=grid_spec,
        out_shape=jax.ShapeDtypeStruct((B, S + 1, D), feature.dtype),
        compiler_params=pltpu.CompilerParams(vmem_limit_bytes=100 * 1024 * 1024),
    )(emb, feature)
